# Initial kernel scaffold; baseline (speedup 1.0000x reference)
import jax
import jax.numpy as jnp
from jax.experimental import pallas as pl


def kernel(x, edge_index, Wl1, bl1, Wr1, Wm1, bm1, Wm2, bm2, Wl2, bl2, Wr2):
    raise NotImplementedError("write your pallas kernel here")


# speedup vs baseline: 7.4780x; 7.4780x over previous
import functools

import jax
import jax.numpy as jnp
from jax import lax
from jax.experimental import pallas as pl
from jax.experimental.pallas import tpu as pltpu
from jax.experimental.pallas import tpu_sc as plsc

N = 10000
E = 320000
D = 128

NC = 2
NS = 16
NW = NC * NS
EPW = E // NW
K = 80
J = EPW // K
JB = 25
NB = J // JB
ZR = 80
NZ = N // ZR
KMAX = -(-NZ // NS)
CW = 16

f32 = jnp.float32


def _make_sc_agg(with_cnt: bool):
    mesh = plsc.VectorSubcoreMesh(core_axis_name="c", subcore_axis_name="s")
    out_type = [jax.ShapeDtypeStruct((NC, N, D), f32)]
    if with_cnt:
        out_type.append(jax.ShapeDtypeStruct((NC, N, CW), f32))
    scratch = [
        pltpu.VMEM_SHARED((N, D), f32),
        pltpu.VMEM((JB, K), jnp.int32),
        pltpu.VMEM((JB, K), jnp.int32),
        pltpu.VMEM((K, D), f32),
        pltpu.VMEM((ZR, D), f32),
        pltpu.VMEM((K,), jnp.int32),
        pltpu.SemaphoreType.DMA((2,)),
    ]
    if with_cnt:
        scratch += [
            pltpu.VMEM_SHARED((N, CW), f32),
            pltpu.VMEM((K, CW), f32),
        ]

    def body(x_hbm, src_hbm, dst_hbm, ones_hbm, zer_hbm, zcnt_hbm,
             *outs_scratch):
        if with_cnt:
            (agg_out, cnt_out, agg_sp, src_v, dst_v, rows_v, zbuf_v, didx_v,
             sems, cnt_sp, ones_v) = outs_scratch
        else:
            (agg_out, agg_sp, src_v, dst_v, rows_v, zbuf_v, didx_v, sems) = \
                outs_scratch
        c = lax.axis_index("c")
        s = lax.axis_index("s")
        wid = c * NS + s

        pltpu.sync_copy(zer_hbm, zbuf_v)
        if with_cnt:
            pltpu.sync_copy(zcnt_hbm, ones_v)

        for k in range(KMAX):
            cid = jnp.minimum(s + NS * k, NZ - 1)
            pltpu.sync_copy(zbuf_v, agg_sp.at[pl.ds(cid * ZR, ZR)])
            if with_cnt:
                pltpu.sync_copy(ones_v, cnt_sp.at[pl.ds(cid * ZR, ZR)])
        if with_cnt:
            pltpu.sync_copy(ones_hbm, ones_v)
        plsc.subcore_barrier()

        for blk in range(NB):
            pltpu.sync_copy(src_hbm.at[wid, blk], src_v)
            pltpu.sync_copy(dst_hbm.at[wid, blk], dst_v)

            def step(j, carry):
                pltpu.async_copy(x_hbm.at[src_v.at[j]], rows_v,
                                 sems.at[0]).wait()
                for i in range(K // 16):
                    didx_v[pl.ds(16 * i, 16)] = dst_v[j, pl.ds(16 * i, 16)]
                pltpu.async_copy(rows_v, agg_sp.at[didx_v], sems.at[1],
                                 add=True).wait()
                if with_cnt:
                    pltpu.async_copy(ones_v, cnt_sp.at[didx_v], sems.at[1],
                                     add=True).wait()
                return carry

            lax.fori_loop(0, JB, step, 0)
        plsc.subcore_barrier()

        for k in range(KMAX):
            cid = jnp.minimum(s + NS * k, NZ - 1)
            sl = pl.ds(cid * ZR, ZR)
            pltpu.sync_copy(agg_sp.at[sl], zbuf_v)
            pltpu.sync_copy(zbuf_v, agg_out.at[c, sl])
            if with_cnt:
                pltpu.sync_copy(cnt_sp.at[sl], ones_v)
                pltpu.sync_copy(ones_v, cnt_out.at[c, sl])

    return pl.kernel(body, out_type=out_type, mesh=mesh,
                     scratch_types=scratch,
                     compiler_params=pltpu.CompilerParams(
                         use_tc_tiling_on_sc=False))


_sc_agg_cnt = _make_sc_agg(True)
_sc_agg = _make_sc_agg(False)


def _mm_t(a, w):
    return lax.dot_general(a, w, (((1,), (1,)), ((), ())),
                           preferred_element_type=f32)


BN = 1000


def _mean_of(agg_ref, cnt_ref):
    agg = agg_ref[0] + agg_ref[1]
    cnt = cnt_ref[0, :, 0:1] + cnt_ref[1, :, 0:1]
    return agg * (1.0 / jnp.maximum(cnt, 1.0))


def _dense1_body(agg_ref, cnt_ref, x_ref, wl_ref, bl_ref, wr_ref,
                 wm1_ref, bm1_ref, wm2_ref, bm2_ref, o_ref):
    mean = _mean_of(agg_ref, cnt_ref)
    h = jnp.maximum(_mm_t(mean, wl_ref[...]) + bl_ref[...]
                    + _mm_t(x_ref[...], wr_ref[...]), 0.0)
    t = jnp.maximum(_mm_t(h, wm1_ref[...]) + bm1_ref[...], 0.0)
    o_ref[...] = _mm_t(t, wm2_ref[...]) + bm2_ref[...]


def _dense2_body(agg_ref, cnt_ref, h_ref, wl_ref, bl_ref, wr_ref, o_ref):
    mean = _mean_of(agg_ref, cnt_ref)
    o_ref[...] = (_mm_t(mean, wl_ref[...]) + bl_ref[...]
                  + _mm_t(h_ref[...], wr_ref[...]))


def _w_spec():
    return pl.BlockSpec((D, D), lambda i: (0, 0))


def _b_spec():
    return pl.BlockSpec((D,), lambda i: (0,))


_ROW_SPEC = pl.BlockSpec((BN, D), lambda i: (i, 0))
_AGG_SPEC = pl.BlockSpec((NC, BN, D), lambda i: (0, i, 0))
_CNT_SPEC = pl.BlockSpec((NC, BN, CW), lambda i: (0, i, 0))

_dense1 = pl.pallas_call(
    _dense1_body,
    grid=(N // BN,),
    in_specs=[_AGG_SPEC, _CNT_SPEC, _ROW_SPEC, _w_spec(), _b_spec(),
              _w_spec(), _w_spec(), _b_spec(), _w_spec(), _b_spec()],
    out_specs=_ROW_SPEC,
    out_shape=jax.ShapeDtypeStruct((N, D), f32),
)

_dense2 = pl.pallas_call(
    _dense2_body,
    grid=(N // BN,),
    in_specs=[_AGG_SPEC, _CNT_SPEC, _ROW_SPEC, _w_spec(), _b_spec(),
              _w_spec()],
    out_specs=_ROW_SPEC,
    out_shape=jax.ShapeDtypeStruct((N, D), f32),
)


def kernel(x, edge_index, Wl1, bl1, Wr1, Wm1, bm1, Wm2, bm2, Wl2, bl2, Wr2):
    src_r = edge_index[0].reshape(NW, NB, JB, K)
    dst_r = edge_index[1].reshape(NW, NB, JB, K)
    ones = jnp.ones((K, CW), f32)
    zer = jnp.zeros((ZR, D), f32)
    zcnt = jnp.zeros((ZR, CW), f32)

    agg1, cnt = _sc_agg_cnt(x, src_r, dst_r, ones, zer, zcnt)
    h2 = _dense1(agg1, cnt, x, Wl1, bl1, Wr1, Wm1, bm1, Wm2, bm2)
    (agg2,) = _sc_agg(h2, src_r, dst_r, ones, zer, zcnt)
    out = _dense2(agg2, cnt, h2, Wl2, bl2, Wr2)
    return out


# speedup vs baseline: 11.4048x; 1.5251x over previous
import functools

import jax
import jax.numpy as jnp
from jax import lax
from jax.experimental import pallas as pl
from jax.experimental.pallas import tpu as pltpu
from jax.experimental.pallas import tpu_sc as plsc

N = 10000
E = 320000
D = 128

NC = 2
NS = 16
NW = NC * NS
EPW = E // NW
K = 80
J = EPW // K
JB = 25
NB = J // JB
ZR = 80
NZ = N // ZR
KMAX = -(-NZ // NS)
CW = 16

f32 = jnp.float32


def _make_sc_agg(with_cnt: bool):
    mesh = plsc.VectorSubcoreMesh(core_axis_name="c", subcore_axis_name="s")
    out_type = [jax.ShapeDtypeStruct((NC, N, D), f32)]
    if with_cnt:
        out_type.append(jax.ShapeDtypeStruct((NC, N, CW), f32))
    scratch = [
        pltpu.VMEM_SHARED((N, D), f32),
        pltpu.VMEM((JB, K), jnp.int32),
        pltpu.VMEM((JB, K), jnp.int32),
        pltpu.VMEM((2, K, D), f32),
        pltpu.VMEM((ZR, D), f32),
        pltpu.SemaphoreType.DMA((2,)),
    ]
    if with_cnt:
        scratch += [
            pltpu.VMEM_SHARED((N, CW), f32),
            pltpu.VMEM((K, CW), f32),
        ]

    def body(x_hbm, src_hbm, dst_hbm, ones_hbm, zer_hbm, zcnt_hbm,
             *outs_scratch):
        if with_cnt:
            (agg_out, cnt_out, agg_sp, src_v, dst_v, rows_v, zbuf_v,
             sems, cnt_sp, ones_v) = outs_scratch
        else:
            (agg_out, agg_sp, src_v, dst_v, rows_v, zbuf_v, sems) = \
                outs_scratch
        c = lax.axis_index("c")
        s = lax.axis_index("s")
        wid = c * NS + s

        pltpu.sync_copy(zer_hbm, zbuf_v)
        if with_cnt:
            pltpu.sync_copy(zcnt_hbm, ones_v)

        for k in range(KMAX):
            cid = jnp.minimum(s + NS * k, NZ - 1)
            pltpu.sync_copy(zbuf_v, agg_sp.at[pl.ds(cid * ZR, ZR)])
            if with_cnt:
                pltpu.sync_copy(ones_v, cnt_sp.at[pl.ds(cid * ZR, ZR)])
        if with_cnt:
            pltpu.sync_copy(ones_hbm, ones_v)
        plsc.subcore_barrier()

        for blk in range(NB):
            pltpu.sync_copy(src_hbm.at[wid, blk], src_v)
            pltpu.sync_copy(dst_hbm.at[wid, blk], dst_v)
            pltpu.async_copy(x_hbm.at[src_v.at[0]], rows_v.at[0], sems.at[0])

            def step(j, carry):
                b = lax.rem(j, 2)
                nb = lax.rem(j + 1, 2)

                @pl.when(j + 1 < JB)
                def _():
                    pltpu.async_copy(x_hbm.at[src_v.at[j + 1]],
                                     rows_v.at[nb], sems.at[nb])

                pltpu.make_async_copy(x_hbm.at[src_v.at[j]], rows_v.at[b],
                                      sems.at[b]).wait()
                pltpu.sync_copy(rows_v.at[b], agg_sp.at[dst_v.at[j]],
                                add=True)
                if with_cnt:
                    pltpu.sync_copy(ones_v, cnt_sp.at[dst_v.at[j]], add=True)
                return carry

            lax.fori_loop(0, JB, step, 0)
        plsc.subcore_barrier()

        for k in range(KMAX):
            cid = jnp.minimum(s + NS * k, NZ - 1)
            sl = pl.ds(cid * ZR, ZR)
            pltpu.sync_copy(agg_sp.at[sl], zbuf_v)
            pltpu.sync_copy(zbuf_v, agg_out.at[c, sl])
            if with_cnt:
                pltpu.sync_copy(cnt_sp.at[sl], ones_v)
                pltpu.sync_copy(ones_v, cnt_out.at[c, sl])

    return pl.kernel(body, out_type=out_type, mesh=mesh,
                     scratch_types=scratch,
                     compiler_params=pltpu.CompilerParams(
                         use_tc_tiling_on_sc=False))


_sc_agg_cnt = _make_sc_agg(True)
_sc_agg = _make_sc_agg(False)


def _mm_t(a, w):
    return lax.dot_general(a, w, (((1,), (1,)), ((), ())),
                           preferred_element_type=f32)


BN = 1000


def _mean_of(agg_ref, cnt_ref):
    agg = agg_ref[0] + agg_ref[1]
    cnt = cnt_ref[0, :, 0:1] + cnt_ref[1, :, 0:1]
    return agg * (1.0 / jnp.maximum(cnt, 1.0))


def _dense1_body(agg_ref, cnt_ref, x_ref, wl_ref, bl_ref, wr_ref,
                 wm1_ref, bm1_ref, wm2_ref, bm2_ref, o_ref):
    mean = _mean_of(agg_ref, cnt_ref)
    h = jnp.maximum(_mm_t(mean, wl_ref[...]) + bl_ref[...]
                    + _mm_t(x_ref[...], wr_ref[...]), 0.0)
    t = jnp.maximum(_mm_t(h, wm1_ref[...]) + bm1_ref[...], 0.0)
    o_ref[...] = _mm_t(t, wm2_ref[...]) + bm2_ref[...]


def _dense2_body(agg_ref, cnt_ref, h_ref, wl_ref, bl_ref, wr_ref, o_ref):
    mean = _mean_of(agg_ref, cnt_ref)
    o_ref[...] = (_mm_t(mean, wl_ref[...]) + bl_ref[...]
                  + _mm_t(h_ref[...], wr_ref[...]))


def _w_spec():
    return pl.BlockSpec((D, D), lambda i: (0, 0))


def _b_spec():
    return pl.BlockSpec((D,), lambda i: (0,))


_ROW_SPEC = pl.BlockSpec((BN, D), lambda i: (i, 0))
_AGG_SPEC = pl.BlockSpec((NC, BN, D), lambda i: (0, i, 0))
_CNT_SPEC = pl.BlockSpec((NC, BN, CW), lambda i: (0, i, 0))

_dense1 = pl.pallas_call(
    _dense1_body,
    grid=(N // BN,),
    in_specs=[_AGG_SPEC, _CNT_SPEC, _ROW_SPEC, _w_spec(), _b_spec(),
              _w_spec(), _w_spec(), _b_spec(), _w_spec(), _b_spec()],
    out_specs=_ROW_SPEC,
    out_shape=jax.ShapeDtypeStruct((N, D), f32),
)

_dense2 = pl.pallas_call(
    _dense2_body,
    grid=(N // BN,),
    in_specs=[_AGG_SPEC, _CNT_SPEC, _ROW_SPEC, _w_spec(), _b_spec(),
              _w_spec()],
    out_specs=_ROW_SPEC,
    out_shape=jax.ShapeDtypeStruct((N, D), f32),
)


def kernel(x, edge_index, Wl1, bl1, Wr1, Wm1, bm1, Wm2, bm2, Wl2, bl2, Wr2):
    src_r = edge_index[0].reshape(NW, NB, JB, K)
    dst_r = edge_index[1].reshape(NW, NB, JB, K)
    ones = jnp.ones((K, CW), f32)
    zer = jnp.zeros((ZR, D), f32)
    zcnt = jnp.zeros((ZR, CW), f32)

    agg1, cnt = _sc_agg_cnt(x, src_r, dst_r, ones, zer, zcnt)
    h2 = _dense1(agg1, cnt, x, Wl1, bl1, Wr1, Wm1, bm1, Wm2, bm2)
    (agg2,) = _sc_agg(h2, src_r, dst_r, ones, zer, zcnt)
    out = _dense2(agg2, cnt, h2, Wl2, bl2, Wr2)
    return out


# speedup vs baseline: 12.0447x; 1.0561x over previous
import functools

import jax
import jax.numpy as jnp
from jax import lax
from jax.experimental import pallas as pl
from jax.experimental.pallas import tpu as pltpu
from jax.experimental.pallas import tpu_sc as plsc

N = 10000
E = 320000
D = 128

NC = 2
NS = 16
NW = NC * NS
EPW = E // NW
K = 125
J = EPW // K
JB = 16
NB = J // JB
ZR = 125
NZ = N // ZR
KMAX = -(-NZ // NS)
CW = 16

f32 = jnp.float32


def _make_sc_agg(with_cnt: bool):
    mesh = plsc.VectorSubcoreMesh(core_axis_name="c", subcore_axis_name="s")
    out_type = [jax.ShapeDtypeStruct((NC, N, D), f32)]
    if with_cnt:
        out_type.append(jax.ShapeDtypeStruct((NC, N, CW), f32))
    scratch = [
        pltpu.VMEM_SHARED((N, D), f32),
        pltpu.VMEM((JB, K), jnp.int32),
        pltpu.VMEM((JB, K), jnp.int32),
        pltpu.VMEM((2, K, D), f32),
        pltpu.SemaphoreType.DMA((2,)),
    ]
    if with_cnt:
        scratch += [
            pltpu.VMEM_SHARED((N, CW), f32),
            pltpu.VMEM((K, CW), f32),
        ]

    def body(x_hbm, src_hbm, dst_hbm, ones_hbm, zer_hbm, zcnt_hbm,
             *outs_scratch):
        if with_cnt:
            (agg_out, cnt_out, agg_sp, src_v, dst_v, rows_v,
             sems, cnt_sp, ones_v) = outs_scratch
        else:
            (agg_out, agg_sp, src_v, dst_v, rows_v, sems) = outs_scratch
        c = lax.axis_index("c")
        s = lax.axis_index("s")
        wid = c * NS + s
        zbuf_v = rows_v.at[0]

        pltpu.sync_copy(zer_hbm, zbuf_v)
        if with_cnt:
            pltpu.sync_copy(zcnt_hbm, ones_v)

        for k in range(KMAX):
            cid = jnp.minimum(s + NS * k, NZ - 1)
            pltpu.sync_copy(zbuf_v, agg_sp.at[pl.ds(cid * ZR, ZR)])
            if with_cnt:
                pltpu.sync_copy(ones_v, cnt_sp.at[pl.ds(cid * ZR, ZR)])
        if with_cnt:
            pltpu.sync_copy(ones_hbm, ones_v)
        plsc.subcore_barrier()

        for blk in range(NB):
            pltpu.sync_copy(src_hbm.at[wid, blk], src_v)
            pltpu.sync_copy(dst_hbm.at[wid, blk], dst_v)
            pltpu.async_copy(x_hbm.at[src_v.at[0]], rows_v.at[0], sems.at[0])

            def step(j, carry):
                b = lax.rem(j, 2)
                nb = lax.rem(j + 1, 2)

                @pl.when(j + 1 < JB)
                def _():
                    pltpu.async_copy(x_hbm.at[src_v.at[j + 1]],
                                     rows_v.at[nb], sems.at[nb])

                pltpu.make_async_copy(x_hbm.at[src_v.at[j]], rows_v.at[b],
                                      sems.at[b]).wait()
                pltpu.sync_copy(rows_v.at[b], agg_sp.at[dst_v.at[j]],
                                add=True)
                if with_cnt:
                    pltpu.sync_copy(ones_v, cnt_sp.at[dst_v.at[j]], add=True)
                return carry

            lax.fori_loop(0, JB, step, 0)
        plsc.subcore_barrier()

        for k in range(KMAX):
            cid = jnp.minimum(s + NS * k, NZ - 1)
            sl = pl.ds(cid * ZR, ZR)
            pltpu.sync_copy(agg_sp.at[sl], zbuf_v)
            pltpu.sync_copy(zbuf_v, agg_out.at[c, sl])
            if with_cnt:
                pltpu.sync_copy(cnt_sp.at[sl], ones_v)
                pltpu.sync_copy(ones_v, cnt_out.at[c, sl])

    return pl.kernel(body, out_type=out_type, mesh=mesh,
                     scratch_types=scratch,
                     compiler_params=pltpu.CompilerParams(
                         use_tc_tiling_on_sc=False))


_sc_agg_cnt = _make_sc_agg(True)
_sc_agg = _make_sc_agg(False)


def _mm_t(a, w):
    return lax.dot_general(a, w, (((1,), (1,)), ((), ())),
                           preferred_element_type=f32)


BN = 1000


def _mean_of(agg_ref, cnt_ref):
    agg = agg_ref[0] + agg_ref[1]
    cnt = cnt_ref[0, :, 0:1] + cnt_ref[1, :, 0:1]
    return agg * (1.0 / jnp.maximum(cnt, 1.0))


def _dense1_body(agg_ref, cnt_ref, x_ref, wl_ref, bl_ref, wr_ref,
                 wm1_ref, bm1_ref, wm2_ref, bm2_ref, o_ref):
    mean = _mean_of(agg_ref, cnt_ref)
    h = jnp.maximum(_mm_t(mean, wl_ref[...]) + bl_ref[...]
                    + _mm_t(x_ref[...], wr_ref[...]), 0.0)
    t = jnp.maximum(_mm_t(h, wm1_ref[...]) + bm1_ref[...], 0.0)
    o_ref[...] = _mm_t(t, wm2_ref[...]) + bm2_ref[...]


def _dense2_body(agg_ref, cnt_ref, h_ref, wl_ref, bl_ref, wr_ref, o_ref):
    mean = _mean_of(agg_ref, cnt_ref)
    o_ref[...] = (_mm_t(mean, wl_ref[...]) + bl_ref[...]
                  + _mm_t(h_ref[...], wr_ref[...]))


def _w_spec():
    return pl.BlockSpec((D, D), lambda i: (0, 0))


def _b_spec():
    return pl.BlockSpec((D,), lambda i: (0,))


_ROW_SPEC = pl.BlockSpec((BN, D), lambda i: (i, 0))
_AGG_SPEC = pl.BlockSpec((NC, BN, D), lambda i: (0, i, 0))
_CNT_SPEC = pl.BlockSpec((NC, BN, CW), lambda i: (0, i, 0))

_dense1 = pl.pallas_call(
    _dense1_body,
    grid=(N // BN,),
    in_specs=[_AGG_SPEC, _CNT_SPEC, _ROW_SPEC, _w_spec(), _b_spec(),
              _w_spec(), _w_spec(), _b_spec(), _w_spec(), _b_spec()],
    out_specs=_ROW_SPEC,
    out_shape=jax.ShapeDtypeStruct((N, D), f32),
)

_dense2 = pl.pallas_call(
    _dense2_body,
    grid=(N // BN,),
    in_specs=[_AGG_SPEC, _CNT_SPEC, _ROW_SPEC, _w_spec(), _b_spec(),
              _w_spec()],
    out_specs=_ROW_SPEC,
    out_shape=jax.ShapeDtypeStruct((N, D), f32),
)


def kernel(x, edge_index, Wl1, bl1, Wr1, Wm1, bm1, Wm2, bm2, Wl2, bl2, Wr2):
    src_r = edge_index[0].reshape(NW, NB, JB, K)
    dst_r = edge_index[1].reshape(NW, NB, JB, K)
    ones = jnp.ones((K, CW), f32)
    zer = jnp.zeros((ZR, D), f32)
    zcnt = jnp.zeros((ZR, CW), f32)

    agg1, cnt = _sc_agg_cnt(x, src_r, dst_r, ones, zer, zcnt)
    h2 = _dense1(agg1, cnt, x, Wl1, bl1, Wr1, Wm1, bm1, Wm2, bm2)
    (agg2,) = _sc_agg(h2, src_r, dst_r, ones, zer, zcnt)
    out = _dense2(agg2, cnt, h2, Wl2, bl2, Wr2)
    return out


# speedup vs baseline: 12.3038x; 1.0215x over previous
import functools

import jax
import jax.numpy as jnp
from jax import lax
from jax.experimental import pallas as pl
from jax.experimental.pallas import tpu as pltpu
from jax.experimental.pallas import tpu_sc as plsc

N = 10000
E = 320000
D = 128

NC = 2
NS = 16
NW = NC * NS
EPW = E // NW
K = 125
J = EPW // K
JB = 16
NB = J // JB
ZR = 125
NZ = N // ZR
KMAX = -(-NZ // NS)
CW = 16

f32 = jnp.float32


def _make_sc_agg(with_cnt: bool):
    mesh = plsc.VectorSubcoreMesh(core_axis_name="c", subcore_axis_name="s")
    out_type = [jax.ShapeDtypeStruct((NC, N, D), f32)]
    if with_cnt:
        out_type.append(jax.ShapeDtypeStruct((NC, N, CW), f32))
    scratch = [
        pltpu.VMEM_SHARED((N, D), f32),
        pltpu.VMEM((JB, K), jnp.int32),
        pltpu.VMEM((JB, K), jnp.int32),
        pltpu.VMEM((2, K, D), f32),
        pltpu.SemaphoreType.DMA((2,)),
    ]
    if with_cnt:
        scratch += [
            pltpu.VMEM_SHARED((N, CW), f32),
            pltpu.VMEM((K, CW), f32),
        ]

    def body(x_hbm, src_hbm, dst_hbm, ones_hbm, zer_hbm, zcnt_hbm,
             *outs_scratch):
        if with_cnt:
            (agg_out, cnt_out, agg_sp, src_v, dst_v, rows_v,
             sems, cnt_sp, ones_v) = outs_scratch
        else:
            (agg_out, agg_sp, src_v, dst_v, rows_v, sems) = outs_scratch
        c = lax.axis_index("c")
        s = lax.axis_index("s")
        wid = c * NS + s
        zbuf_v = rows_v.at[0]

        pltpu.sync_copy(zer_hbm, zbuf_v)
        if with_cnt:
            pltpu.sync_copy(zcnt_hbm, ones_v)

        for k in range(KMAX):
            cid = jnp.minimum(s + NS * k, NZ - 1)
            pltpu.sync_copy(zbuf_v, agg_sp.at[pl.ds(cid * ZR, ZR)])
            if with_cnt:
                pltpu.sync_copy(ones_v, cnt_sp.at[pl.ds(cid * ZR, ZR)])
        if with_cnt:
            pltpu.sync_copy(ones_hbm, ones_v)
        plsc.subcore_barrier()

        for blk in range(NB):
            pltpu.sync_copy(src_hbm.at[wid, blk], src_v)
            pltpu.sync_copy(dst_hbm.at[wid, blk], dst_v)
            pltpu.async_copy(x_hbm.at[src_v.at[0]], rows_v.at[0], sems.at[0])

            def step(j, carry):
                b = lax.rem(j, 2)
                nb = lax.rem(j + 1, 2)

                @pl.when(j + 1 < JB)
                def _():
                    pltpu.async_copy(x_hbm.at[src_v.at[j + 1]],
                                     rows_v.at[nb], sems.at[nb])

                pltpu.make_async_copy(x_hbm.at[src_v.at[j]], rows_v.at[b],
                                      sems.at[b]).wait()
                pltpu.sync_copy(rows_v.at[b], agg_sp.at[dst_v.at[j]],
                                add=True)
                if with_cnt:
                    pltpu.sync_copy(ones_v, cnt_sp.at[dst_v.at[j]], add=True)
                return carry

            lax.fori_loop(0, JB, step, 0)
        plsc.subcore_barrier()

        for k in range(KMAX):
            cid = jnp.minimum(s + NS * k, NZ - 1)
            sl = pl.ds(cid * ZR, ZR)
            pltpu.sync_copy(agg_sp.at[sl], zbuf_v)
            pltpu.sync_copy(zbuf_v, agg_out.at[c, sl])
            if with_cnt:
                pltpu.sync_copy(cnt_sp.at[sl], ones_v)
                pltpu.sync_copy(ones_v, cnt_out.at[c, sl])

    return pl.kernel(body, out_type=out_type, mesh=mesh,
                     scratch_types=scratch,
                     compiler_params=pltpu.CompilerParams(
                         use_tc_tiling_on_sc=False))


_sc_agg_cnt = _make_sc_agg(True)
_sc_agg = _make_sc_agg(False)


def _mm_t(a, w):
    return lax.dot_general(a, w, (((1,), (1,)), ((), ())),
                           preferred_element_type=f32)


BN = 2000


def _mean_of(agg_ref, cnt_ref):
    agg = agg_ref[0] + agg_ref[1]
    cnt = cnt_ref[0, :, 0:1] + cnt_ref[1, :, 0:1]
    return agg * (1.0 / jnp.maximum(cnt, 1.0))


def _dense1_body(agg_ref, cnt_ref, x_ref, wl_ref, bl_ref, wr_ref,
                 wm1_ref, bm1_ref, wm2_ref, bm2_ref, o_ref):
    mean = _mean_of(agg_ref, cnt_ref)
    h = jnp.maximum(_mm_t(mean, wl_ref[...]) + bl_ref[...]
                    + _mm_t(x_ref[...], wr_ref[...]), 0.0)
    t = jnp.maximum(_mm_t(h, wm1_ref[...]) + bm1_ref[...], 0.0)
    o_ref[...] = _mm_t(t, wm2_ref[...]) + bm2_ref[...]


def _dense2_body(agg_ref, cnt_ref, h_ref, wl_ref, bl_ref, wr_ref, o_ref):
    mean = _mean_of(agg_ref, cnt_ref)
    o_ref[...] = (_mm_t(mean, wl_ref[...]) + bl_ref[...]
                  + _mm_t(h_ref[...], wr_ref[...]))


def _w_spec():
    return pl.BlockSpec((D, D), lambda i: (0, 0))


def _b_spec():
    return pl.BlockSpec((D,), lambda i: (0,))


_ROW_SPEC = pl.BlockSpec((BN, D), lambda i: (i, 0))
_AGG_SPEC = pl.BlockSpec((NC, BN, D), lambda i: (0, i, 0))
_CNT_SPEC = pl.BlockSpec((NC, BN, CW), lambda i: (0, i, 0))

_dense1 = pl.pallas_call(
    _dense1_body,
    grid=(N // BN,),
    in_specs=[_AGG_SPEC, _CNT_SPEC, _ROW_SPEC, _w_spec(), _b_spec(),
              _w_spec(), _w_spec(), _b_spec(), _w_spec(), _b_spec()],
    out_specs=_ROW_SPEC,
    out_shape=jax.ShapeDtypeStruct((N, D), f32),
)

_dense2 = pl.pallas_call(
    _dense2_body,
    grid=(N // BN,),
    in_specs=[_AGG_SPEC, _CNT_SPEC, _ROW_SPEC, _w_spec(), _b_spec(),
              _w_spec()],
    out_specs=_ROW_SPEC,
    out_shape=jax.ShapeDtypeStruct((N, D), f32),
)


def kernel(x, edge_index, Wl1, bl1, Wr1, Wm1, bm1, Wm2, bm2, Wl2, bl2, Wr2):
    src_r = edge_index[0].reshape(NW, NB, JB, K)
    dst_r = edge_index[1].reshape(NW, NB, JB, K)
    ones = jnp.ones((K, CW), f32)
    zer = jnp.zeros((ZR, D), f32)
    zcnt = jnp.zeros((ZR, CW), f32)

    agg1, cnt = _sc_agg_cnt(x, src_r, dst_r, ones, zer, zcnt)
    h2 = _dense1(agg1, cnt, x, Wl1, bl1, Wr1, Wm1, bm1, Wm2, bm2)
    (agg2,) = _sc_agg(h2, src_r, dst_r, ones, zer, zcnt)
    out = _dense2(agg2, cnt, h2, Wl2, bl2, Wr2)
    return out


# speedup vs baseline: 12.8581x; 1.0450x over previous
import functools

import jax
import jax.numpy as jnp
from jax import lax
from jax.experimental import pallas as pl
from jax.experimental.pallas import tpu as pltpu
from jax.experimental.pallas import tpu_sc as plsc

N = 10000
E = 320000
D = 128

NC = 2
NS = 16
NW = NC * NS
EPW = E // NW
K = 125
J = EPW // K
JB = 16
NB = J // JB
ZR = 125
NZ = N // ZR
KMAX = -(-NZ // NS)
CW = 8

f32 = jnp.float32


def _make_sc_agg(with_cnt: bool):
    mesh = plsc.VectorSubcoreMesh(core_axis_name="c", subcore_axis_name="s")
    out_type = [jax.ShapeDtypeStruct((NC, N, D), f32)]
    if with_cnt:
        out_type.append(jax.ShapeDtypeStruct((NC, N, CW), f32))
    scratch = [
        pltpu.VMEM_SHARED((N, D), f32),
        pltpu.VMEM((2, JB, K), jnp.int32),
        pltpu.VMEM((2, JB, K), jnp.int32),
        pltpu.VMEM((2, K, D), f32),
        pltpu.SemaphoreType.DMA((4,)),
    ]
    if with_cnt:
        scratch += [
            pltpu.VMEM_SHARED((N, CW), f32),
            pltpu.VMEM((K, CW), f32),
        ]

    def body(x_hbm, src_hbm, dst_hbm, ones_hbm, zer_hbm, zcnt_hbm,
             *outs_scratch):
        if with_cnt:
            (agg_out, cnt_out, agg_sp, src_v, dst_v, rows_v,
             sems, cnt_sp, ones_v) = outs_scratch
        else:
            (agg_out, agg_sp, src_v, dst_v, rows_v, sems) = outs_scratch
        c = lax.axis_index("c")
        s = lax.axis_index("s")
        wid = c * NS + s
        zbuf_v = rows_v.at[0]

        pltpu.async_copy(src_hbm.at[wid, 0], src_v.at[0], sems.at[2])
        pltpu.async_copy(dst_hbm.at[wid, 0], dst_v.at[0], sems.at[3])

        pltpu.sync_copy(zer_hbm, zbuf_v)
        if with_cnt:
            pltpu.sync_copy(zcnt_hbm, ones_v)

        for k in range(KMAX):
            cid = jnp.minimum(s + NS * k, NZ - 1)
            pltpu.sync_copy(zbuf_v, agg_sp.at[pl.ds(cid * ZR, ZR)])
            if with_cnt:
                pltpu.sync_copy(ones_v, cnt_sp.at[pl.ds(cid * ZR, ZR)])
        if with_cnt:
            pltpu.sync_copy(ones_hbm, ones_v)
        plsc.subcore_barrier()

        for blk in range(NB):
            bb = blk % 2
            pltpu.make_async_copy(src_hbm.at[wid, blk], src_v.at[bb],
                                  sems.at[2]).wait()
            pltpu.make_async_copy(dst_hbm.at[wid, blk], dst_v.at[bb],
                                  sems.at[3]).wait()
            if blk + 1 < NB:
                pltpu.async_copy(src_hbm.at[wid, blk + 1],
                                 src_v.at[1 - bb], sems.at[2])
                pltpu.async_copy(dst_hbm.at[wid, blk + 1],
                                 dst_v.at[1 - bb], sems.at[3])
            pltpu.async_copy(x_hbm.at[src_v.at[bb, 0]], rows_v.at[0],
                             sems.at[0])

            def step(j, carry):
                b = lax.rem(j, 2)
                nb = lax.rem(j + 1, 2)

                @pl.when(j + 1 < JB)
                def _():
                    pltpu.async_copy(x_hbm.at[src_v.at[bb, j + 1]],
                                     rows_v.at[nb], sems.at[nb])

                pltpu.make_async_copy(x_hbm.at[src_v.at[bb, j]],
                                      rows_v.at[b], sems.at[b]).wait()
                pltpu.sync_copy(rows_v.at[b], agg_sp.at[dst_v.at[bb, j]],
                                add=True)
                if with_cnt:
                    pltpu.sync_copy(ones_v, cnt_sp.at[dst_v.at[bb, j]],
                                    add=True)
                return carry

            lax.fori_loop(0, JB, step, 0)
        plsc.subcore_barrier()

        for k in range(KMAX):
            cid = jnp.minimum(s + NS * k, NZ - 1)
            sl = pl.ds(cid * ZR, ZR)
            pltpu.sync_copy(agg_sp.at[sl], zbuf_v)
            pltpu.sync_copy(zbuf_v, agg_out.at[c, sl])
            if with_cnt:
                pltpu.sync_copy(cnt_sp.at[sl], ones_v)
                pltpu.sync_copy(ones_v, cnt_out.at[c, sl])

    return pl.kernel(body, out_type=out_type, mesh=mesh,
                     scratch_types=scratch,
                     compiler_params=pltpu.CompilerParams(
                         use_tc_tiling_on_sc=False))


_sc_agg_cnt = _make_sc_agg(True)
_sc_agg = _make_sc_agg(False)


def _mm_t(a, w):
    return lax.dot_general(a, w, (((1,), (1,)), ((), ())),
                           preferred_element_type=f32)


BN = 2000


def _mean_of(agg_ref, cnt_ref):
    agg = agg_ref[0] + agg_ref[1]
    cnt = cnt_ref[0, :, 0:1] + cnt_ref[1, :, 0:1]
    return agg * (1.0 / jnp.maximum(cnt, 1.0))


def _dense1_body(agg_ref, cnt_ref, x_ref, wl_ref, bl_ref, wr_ref,
                 wm1_ref, bm1_ref, wm2_ref, bm2_ref, o_ref):
    mean = _mean_of(agg_ref, cnt_ref)
    h = jnp.maximum(_mm_t(mean, wl_ref[...]) + bl_ref[...]
                    + _mm_t(x_ref[...], wr_ref[...]), 0.0)
    t = jnp.maximum(_mm_t(h, wm1_ref[...]) + bm1_ref[...], 0.0)
    o_ref[...] = _mm_t(t, wm2_ref[...]) + bm2_ref[...]


def _dense2_body(agg_ref, cnt_ref, h_ref, wl_ref, bl_ref, wr_ref, o_ref):
    mean = _mean_of(agg_ref, cnt_ref)
    o_ref[...] = (_mm_t(mean, wl_ref[...]) + bl_ref[...]
                  + _mm_t(h_ref[...], wr_ref[...]))


def _w_spec():
    return pl.BlockSpec((D, D), lambda i: (0, 0))


def _b_spec():
    return pl.BlockSpec((D,), lambda i: (0,))


_ROW_SPEC = pl.BlockSpec((BN, D), lambda i: (i, 0))
_AGG_SPEC = pl.BlockSpec((NC, BN, D), lambda i: (0, i, 0))
_CNT_SPEC = pl.BlockSpec((NC, BN, CW), lambda i: (0, i, 0))

_dense1 = pl.pallas_call(
    _dense1_body,
    grid=(N // BN,),
    in_specs=[_AGG_SPEC, _CNT_SPEC, _ROW_SPEC, _w_spec(), _b_spec(),
              _w_spec(), _w_spec(), _b_spec(), _w_spec(), _b_spec()],
    out_specs=_ROW_SPEC,
    out_shape=jax.ShapeDtypeStruct((N, D), f32),
)

_dense2 = pl.pallas_call(
    _dense2_body,
    grid=(N // BN,),
    in_specs=[_AGG_SPEC, _CNT_SPEC, _ROW_SPEC, _w_spec(), _b_spec(),
              _w_spec()],
    out_specs=_ROW_SPEC,
    out_shape=jax.ShapeDtypeStruct((N, D), f32),
)


def kernel(x, edge_index, Wl1, bl1, Wr1, Wm1, bm1, Wm2, bm2, Wl2, bl2, Wr2):
    src_r = edge_index[0].reshape(NW, NB, JB, K)
    dst_r = edge_index[1].reshape(NW, NB, JB, K)
    ones = jnp.ones((K, CW), f32)
    zer = jnp.zeros((ZR, D), f32)
    zcnt = jnp.zeros((ZR, CW), f32)

    agg1, cnt = _sc_agg_cnt(x, src_r, dst_r, ones, zer, zcnt)
    h2 = _dense1(agg1, cnt, x, Wl1, bl1, Wr1, Wm1, bm1, Wm2, bm2)
    (agg2,) = _sc_agg(h2, src_r, dst_r, ones, zer, zcnt)
    out = _dense2(agg2, cnt, h2, Wl2, bl2, Wr2)
    return out


# speedup vs baseline: 12.8614x; 1.0003x over previous
import jax
import jax.numpy as jnp
from jax import lax
from jax.experimental import pallas as pl
from jax.experimental.pallas import tpu as pltpu
from jax.experimental.pallas import tpu_sc as plsc

N = 10000
E = 320000
D = 128

NC = 2
NS = 16
NW = NC * NS
EPW = E // NW
K = 125
J = EPW // K
JB = 16
NB = J // JB
ZR = 125
NZ = N // ZR
KMAX = -(-NZ // NS)
CW = 8

f32 = jnp.float32


def _make_sc_agg(with_cnt: bool):
    mesh = plsc.VectorSubcoreMesh(core_axis_name="c", subcore_axis_name="s")
    out_type = [jax.ShapeDtypeStruct((NC, N, D), f32)]
    if with_cnt:
        out_type.append(jax.ShapeDtypeStruct((NC, N, CW), f32))
    scratch = [
        pltpu.VMEM_SHARED((N, D), f32),
        pltpu.VMEM((2, JB, K), jnp.int32),
        pltpu.VMEM((2, JB, K), jnp.int32),
        pltpu.VMEM((2, K, D), f32),
        pltpu.SemaphoreType.DMA((4,)),
    ]
    if with_cnt:
        scratch += [
            pltpu.VMEM_SHARED((N, CW), f32),
            pltpu.VMEM((K, CW), f32),
        ]

    def body(x_hbm, src_hbm, dst_hbm, ones_hbm, zer_hbm, zcnt_hbm,
             *outs_scratch):
        if with_cnt:
            (agg_out, cnt_out, agg_sp, src_v, dst_v, rows_v,
             sems, cnt_sp, ones_v) = outs_scratch
        else:
            (agg_out, agg_sp, src_v, dst_v, rows_v, sems) = outs_scratch
        c = lax.axis_index("c")
        s = lax.axis_index("s")
        wid = c * NS + s
        zbuf_v = rows_v.at[0]

        pltpu.async_copy(src_hbm.at[wid, 0], src_v.at[0], sems.at[2])
        pltpu.async_copy(dst_hbm.at[wid, 0], dst_v.at[0], sems.at[3])

        pltpu.sync_copy(zer_hbm, zbuf_v)
        if with_cnt:
            pltpu.sync_copy(zcnt_hbm, ones_v)

        for k in range(KMAX):
            cid = jnp.minimum(s + NS * k, NZ - 1)
            pltpu.sync_copy(zbuf_v, agg_sp.at[pl.ds(cid * ZR, ZR)])
            if with_cnt:
                pltpu.sync_copy(ones_v, cnt_sp.at[pl.ds(cid * ZR, ZR)])
        if with_cnt:
            pltpu.sync_copy(ones_hbm, ones_v)
        plsc.subcore_barrier()

        for blk in range(NB):
            bb = blk % 2
            pltpu.make_async_copy(src_hbm.at[wid, blk], src_v.at[bb],
                                  sems.at[2]).wait()
            pltpu.make_async_copy(dst_hbm.at[wid, blk], dst_v.at[bb],
                                  sems.at[3]).wait()
            if blk + 1 < NB:
                pltpu.async_copy(src_hbm.at[wid, blk + 1],
                                 src_v.at[1 - bb], sems.at[2])
                pltpu.async_copy(dst_hbm.at[wid, blk + 1],
                                 dst_v.at[1 - bb], sems.at[3])
            pltpu.async_copy(x_hbm.at[src_v.at[bb, 0]], rows_v.at[0],
                             sems.at[0])

            def step(j, carry):
                b = lax.rem(j, 2)
                nb = lax.rem(j + 1, 2)

                @pl.when(j + 1 < JB)
                def _():
                    pltpu.async_copy(x_hbm.at[src_v.at[bb, j + 1]],
                                     rows_v.at[nb], sems.at[nb])

                pltpu.make_async_copy(x_hbm.at[src_v.at[bb, j]],
                                      rows_v.at[b], sems.at[b]).wait()
                pltpu.sync_copy(rows_v.at[b], agg_sp.at[dst_v.at[bb, j]],
                                add=True)
                if with_cnt:
                    pltpu.sync_copy(ones_v, cnt_sp.at[dst_v.at[bb, j]],
                                    add=True)
                return carry

            lax.fori_loop(0, JB, step, 0)
        plsc.subcore_barrier()

        for k in range(KMAX):
            cid = jnp.minimum(s + NS * k, NZ - 1)
            sl = pl.ds(cid * ZR, ZR)
            pltpu.sync_copy(agg_sp.at[sl], zbuf_v)
            pltpu.sync_copy(zbuf_v, agg_out.at[c, sl])
            if with_cnt:
                pltpu.sync_copy(cnt_sp.at[sl], ones_v)
                pltpu.sync_copy(ones_v, cnt_out.at[c, sl])

    return pl.kernel(body, out_type=out_type, mesh=mesh,
                     scratch_types=scratch,
                     compiler_params=pltpu.CompilerParams(
                         use_tc_tiling_on_sc=False))


_sc_agg_cnt = _make_sc_agg(True)
_sc_agg = _make_sc_agg(False)


def _mm_t(a, w):
    return lax.dot_general(a, w, (((1,), (1,)), ((), ())),
                           preferred_element_type=f32)


BN = 2000


def _mean_of(agg_ref, cnt_ref):
    agg = agg_ref[0] + agg_ref[1]
    cnt = cnt_ref[0, :, 0:1] + cnt_ref[1, :, 0:1]
    return agg * (1.0 / jnp.maximum(cnt, 1.0))


def _dense1_body(agg_ref, cnt_ref, x_ref, wl_ref, bl_ref, wr_ref,
                 wm1_ref, bm1_ref, wm2_ref, bm2_ref, o_ref):
    mean = _mean_of(agg_ref, cnt_ref)
    h = jnp.maximum(_mm_t(mean, wl_ref[...]) + bl_ref[...]
                    + _mm_t(x_ref[...], wr_ref[...]), 0.0)
    t = jnp.maximum(_mm_t(h, wm1_ref[...]) + bm1_ref[...], 0.0)
    o_ref[...] = _mm_t(t, wm2_ref[...]) + bm2_ref[...]


def _dense2_body(agg_ref, cnt_ref, h_ref, wl_ref, bl_ref, wr_ref, o_ref):
    mean = _mean_of(agg_ref, cnt_ref)
    o_ref[...] = (_mm_t(mean, wl_ref[...]) + bl_ref[...]
                  + _mm_t(h_ref[...], wr_ref[...]))


def _w_spec():
    return pl.BlockSpec((D, D), lambda i: (0, 0))


def _b_spec():
    return pl.BlockSpec((D,), lambda i: (0,))


_ROW_SPEC = pl.BlockSpec((BN, D), lambda i: (i, 0))
_AGG_SPEC = pl.BlockSpec((NC, BN, D), lambda i: (0, i, 0))
_CNT_SPEC = pl.BlockSpec((NC, BN, CW), lambda i: (0, i, 0))

_dense1 = pl.pallas_call(
    _dense1_body,
    grid=(N // BN,),
    in_specs=[_AGG_SPEC, _CNT_SPEC, _ROW_SPEC, _w_spec(), _b_spec(),
              _w_spec(), _w_spec(), _b_spec(), _w_spec(), _b_spec()],
    out_specs=_ROW_SPEC,
    out_shape=jax.ShapeDtypeStruct((N, D), f32),
)

_dense2 = pl.pallas_call(
    _dense2_body,
    grid=(N // BN,),
    in_specs=[_AGG_SPEC, _CNT_SPEC, _ROW_SPEC, _w_spec(), _b_spec(),
              _w_spec()],
    out_specs=_ROW_SPEC,
    out_shape=jax.ShapeDtypeStruct((N, D), f32),
)


def kernel(x, edge_index, Wl1, bl1, Wr1, Wm1, bm1, Wm2, bm2, Wl2, bl2, Wr2):
    src_r = edge_index[0].reshape(NW, NB, JB, K)
    dst_r = edge_index[1].reshape(NW, NB, JB, K)
    ones = jnp.ones((K, CW), f32)
    zer = jnp.zeros((ZR, D), f32)
    zcnt = jnp.zeros((ZR, CW), f32)

    agg1, cnt = _sc_agg_cnt(x, src_r, dst_r, ones, zer, zcnt)
    h2 = _dense1(agg1, cnt, x, Wl1, bl1, Wr1, Wm1, bm1, Wm2, bm2)
    (agg2,) = _sc_agg(h2, src_r, dst_r, ones, zer, zcnt)
    out = _dense2(agg2, cnt, h2, Wl2, bl2, Wr2)
    return out
